# baseline (speedup 1.0000x reference)
import jax
import jax.numpy as jnp
from jax.experimental import pallas as pl
from jax.experimental.pallas import tpu as pltpu

_H0, _W0 = 56, 120
_N_CAM, _M_HEADS, _HG, _WG = 6, 4, 25, 25
_P = _HG * _WG
_K = _N_CAM * _H0 * _W0
_PB = 5
_NRB = _P // _PB
_NT = (_NRB + 2) // 3
_TSTEPS = _NT // 2


def _argmax_body(a0_ref, a1_ref, a2_ref, ood_ref, val_ref):
    nr = _PB * 8
    ki = jax.lax.broadcasted_iota(jnp.int32, (nr, _K), 1)
    rmask = (jax.lax.broadcasted_iota(jnp.int32, (nr, 1), 0) % 8) < 4

    for j, att_ref in enumerate((a0_ref, a1_ref, a2_ref)):
        x = att_ref[...]
        z = x + jnp.roll(x, -1, axis=0)
        z = z + jnp.roll(z, -2, axis=0)
        bmax = jnp.max(z, axis=-1, keepdims=True)
        cand = jnp.where(z == bmax, ki, _K)
        bidx = jnp.min(cand, axis=-1, keepdims=True)
        ood40 = jnp.where(rmask, ood_ref[0], ood_ref[1])
        bval = jnp.sum(jnp.where(cand == bidx, ood40, 0.0),
                       axis=-1, keepdims=True)
        val_ref[j] = bval


def _bce_body(mask_ref, y_ref, out_ref):
    m = mask_ref[0]
    t = y_ref[0, 0]
    logp = jnp.maximum(jnp.log(m), -100.0)
    log1mp = jnp.maximum(jnp.log1p(-m), -100.0)

    cell = jax.lax.broadcasted_iota(jnp.int32, (_WG, 8 * _WG), 1) // 8
    row = jax.lax.broadcasted_iota(jnp.int32, (_WG, 8 * _WG), 0)
    g = (cell == row).astype(jnp.float32)
    logp_w = jnp.dot(logp, g, preferred_element_type=jnp.float32)
    log1mp_w = jnp.dot(log1mp, g, preferred_element_type=jnp.float32)
    tc = jnp.dot(g, t, preferred_element_type=jnp.float32)
    out_ref[...] = -(jnp.sum(tc * logp_w)
                     + jnp.sum((8.0 - tc) * log1mp_w))[None, None, None]


def kernel(alpha, y, ood, ood_cam, att0, att1):
    B = y.shape[0]
    att2d = att0.transpose(1, 0, 2).reshape(_P * 2 * _M_HEADS, _K)
    ood_flat = ood_cam.reshape(B, 1, _K)

    val = pl.pallas_call(
        _argmax_body,
        grid=(2, _TSTEPS),
        in_specs=[
            pl.BlockSpec((_PB * 8, _K),
                         (lambda c, k, j=j: (
                             jnp.minimum(3 * jnp.minimum(c * _TSTEPS + k,
                                                         _NT - 1) + j,
                                         _NRB - 1), 0)))
            for j in range(3)
        ] + [
            pl.BlockSpec((B, 1, _K), lambda c, k: (0, 0, 0)),
        ],
        out_specs=pl.BlockSpec((3, _PB * 8, 1),
                               lambda c, k: (jnp.minimum(c * _TSTEPS + k,
                                                         _NT - 1), 0, 0)),
        out_shape=jax.ShapeDtypeStruct((3 * _NT, _PB * 8, 1), jnp.float32),
        compiler_params=pltpu.CompilerParams(
            dimension_semantics=("parallel", "arbitrary"),
            vmem_limit_bytes=60000 * 1024,
        ),
    )(att2d, att2d, att2d, ood_flat)

    v = val[:_NRB, :, 0].reshape(_NRB, _PB, 8)
    mask = jnp.stack([v[:, :, 0], v[:, :, 4]]).reshape(B, _HG, _WG)

    out = pl.pallas_call(
        _bce_body,
        grid=(B,),
        in_specs=[
            pl.BlockSpec((1, _HG, _WG), lambda b: (b, 0, 0)),
            pl.BlockSpec((1, 1, 8 * _HG, 8 * _WG), lambda b: (b, 0, 0, 0)),
        ],
        out_specs=pl.BlockSpec((1, 1, 1), lambda b: (b, 0, 0)),
        out_shape=jax.ShapeDtypeStruct((B, 1, 1), jnp.float32),
        compiler_params=pltpu.CompilerParams(
            dimension_semantics=("parallel",),
        ),
    )(mask, y)

    return out.sum() / (B * 8 * _HG * 8 * _WG)


# speedup vs baseline: 2.2626x; 2.2626x over previous
import jax
import jax.numpy as jnp
from jax.experimental import pallas as pl
from jax.experimental.pallas import tpu as pltpu

_H0, _W0 = 56, 120
_N_CAM, _M_HEADS, _HG, _WG = 6, 4, 25, 25
_P = _HG * _WG
_K = _N_CAM * _H0 * _W0
_PB = 5
_NRB = _P // _PB
_NT = (_NRB + 2) // 3
_TSTEPS = _NT // 2


def _argmax_body(a0_ref, a1_ref, a2_ref, ood_ref, val_ref):
    rr = jax.lax.broadcasted_iota(jnp.int32, (16, _PB * 8), 0)
    cc = jax.lax.broadcasted_iota(jnp.int32, (16, _PB * 8), 1)
    sel = ((cc // 8 == rr % 8) & ((cc % 8) // 4 == rr // 8)
           & (rr % 8 < _PB)).astype(jnp.float32)
    ki = jax.lax.broadcasted_iota(jnp.int32, (8, _K), 1)

    for j, att_ref in enumerate((a0_ref, a1_ref, a2_ref)):
        x = att_ref[...]
        xr = x - x.astype(jnp.bfloat16).astype(jnp.float32)
        s16 = (jnp.dot(sel, x, preferred_element_type=jnp.float32)
               + jnp.dot(sel, xr, preferred_element_type=jnp.float32))
        for b in range(2):
            s = s16[8 * b:8 * b + 8]
            bmax = jnp.max(s, axis=-1, keepdims=True)
            cand = jnp.where(s == bmax, ki, _K)
            bidx = jnp.min(cand, axis=-1, keepdims=True)
            bval = jnp.sum(jnp.where(cand == bidx, ood_ref[b], 0.0),
                           axis=-1)
            val_ref[j, b] = bval[0:_PB]


def _bce_body(mask_ref, y_ref, out_ref):
    m = mask_ref[0]
    t = y_ref[0, 0]
    logp = jnp.maximum(jnp.log(m), -100.0)
    log1mp = jnp.maximum(jnp.log1p(-m), -100.0)

    cell = jax.lax.broadcasted_iota(jnp.int32, (_WG, 8 * _WG), 1) // 8
    row = jax.lax.broadcasted_iota(jnp.int32, (_WG, 8 * _WG), 0)
    g = (cell == row).astype(jnp.float32)
    logp_w = jnp.dot(logp, g, preferred_element_type=jnp.float32)
    log1mp_w = jnp.dot(log1mp, g, preferred_element_type=jnp.float32)
    tc = jnp.dot(g, t, preferred_element_type=jnp.float32)
    out_ref[...] = -(jnp.sum(tc * logp_w)
                     + jnp.sum((8.0 - tc) * log1mp_w))[None, None, None]


def kernel(alpha, y, ood, ood_cam, att0, att1):
    B = y.shape[0]
    att2d = att0.transpose(1, 0, 2).reshape(_P * 2 * _M_HEADS, _K)
    ood_flat = ood_cam.reshape(B, 1, _K)

    val = pl.pallas_call(
        _argmax_body,
        grid=(2, _TSTEPS),
        in_specs=[
            pl.BlockSpec((_PB * 8, _K),
                         (lambda c, k, j=j: (
                             jnp.minimum(3 * jnp.minimum(c * _TSTEPS + k,
                                                         _NT - 1) + j,
                                         _NRB - 1), 0)))
            for j in range(3)
        ] + [
            pl.BlockSpec((B, 1, _K), lambda c, k: (0, 0, 0)),
        ],
        out_specs=pl.BlockSpec((3, B, _PB),
                               lambda c, k: (jnp.minimum(c * _TSTEPS + k,
                                                         _NT - 1), 0, 0)),
        out_shape=jax.ShapeDtypeStruct((3 * _NT, B, _PB), jnp.float32),
        compiler_params=pltpu.CompilerParams(
            dimension_semantics=("parallel", "arbitrary"),
            vmem_limit_bytes=60000 * 1024,
        ),
    )(att2d, att2d, att2d, ood_flat)

    mask = val[:_NRB].transpose(1, 0, 2).reshape(B, _HG, _WG)

    out = pl.pallas_call(
        _bce_body,
        grid=(B,),
        in_specs=[
            pl.BlockSpec((1, _HG, _WG), lambda b: (b, 0, 0)),
            pl.BlockSpec((1, 1, 8 * _HG, 8 * _WG), lambda b: (b, 0, 0, 0)),
        ],
        out_specs=pl.BlockSpec((1, 1, 1), lambda b: (b, 0, 0)),
        out_shape=jax.ShapeDtypeStruct((B, 1, 1), jnp.float32),
        compiler_params=pltpu.CompilerParams(
            dimension_semantics=("parallel",),
        ),
    )(mask, y)

    return out.sum() / (B * 8 * _HG * 8 * _WG)
